# Initial kernel scaffold; baseline (speedup 1.0000x reference)
import jax
import jax.numpy as jnp
from jax.experimental import pallas as pl


def kernel(tokens, table):
    raise NotImplementedError("write your pallas kernel here")


# speedup vs baseline: 5.8901x; 5.8901x over previous
import functools
import math

import jax
import jax.numpy as jnp
from jax import lax
from jax.experimental import pallas as pl
from jax.experimental.pallas import tpu as pltpu
from jax.experimental.pallas import tpu_sc as plsc

VOCAB = 100000
D = 128
SCALE = math.sqrt(float(D))

_INFO = plsc.get_sparse_core_info()
_NC = _INFO.num_cores
_NS = _INFO.num_subcores
_NW = _NC * _NS

CHUNK = 128


def _scale_body(t_ref, o_ref):
    o_ref[...] = t_ref[...] * SCALE


def _scaled_table(table):
    rows_per_blk = 5000
    grid = VOCAB // rows_per_blk
    return pl.pallas_call(
        _scale_body,
        out_shape=jax.ShapeDtypeStruct((VOCAB, D), jnp.float32),
        grid=(grid,),
        in_specs=[pl.BlockSpec((rows_per_blk, D), lambda i: (i, 0))],
        out_specs=pl.BlockSpec((rows_per_blk, D), lambda i: (i, 0)),
    )(table)


@functools.cache
def _gather_kernel(B):
    n_chunks = B // _NW // CHUNK
    per_w = n_chunks * CHUNK
    mesh = plsc.VectorSubcoreMesh(core_axis_name="c", subcore_axis_name="s")

    @functools.partial(
        pl.kernel,
        mesh=mesh,
        out_type=jax.ShapeDtypeStruct((B, D), jnp.float32),
        scratch_types=[
            pltpu.VMEM((n_chunks, CHUNK), jnp.int32),
            pltpu.VMEM((CHUNK, D), jnp.float32),
            pltpu.SemaphoreType.DMA,
        ],
    )
    def k(idx_hbm, table_hbm, out_hbm, idx_v, rows_v, sem):
        wid = lax.axis_index("s") * _NC + lax.axis_index("c")
        pltpu.sync_copy(idx_hbm.at[wid], idx_v)
        base = wid * per_w

        def body(g, carry):
            pltpu.async_copy(table_hbm.at[idx_v.at[g]], rows_v, sem).wait()
            pltpu.sync_copy(rows_v, out_hbm.at[pl.ds(base + g * CHUNK, CHUNK)])
            return carry

        lax.fori_loop(0, n_chunks, body, 0)

    return k


def kernel(tokens, table):
    b, h = tokens.shape
    B = b * h
    idx = tokens.reshape(_NW, B // _NW // CHUNK, CHUNK)
    scaled = _scaled_table(table)
    out = _gather_kernel(B)(idx, scaled)
    return out.reshape(b, h, D)


# speedup vs baseline: 8.2948x; 1.4082x over previous
import functools
import math

import jax
import jax.numpy as jnp
from jax import lax
from jax.experimental import pallas as pl
from jax.experimental.pallas import tpu as pltpu
from jax.experimental.pallas import tpu_sc as plsc

VOCAB = 100000
D = 128
SCALE = math.sqrt(float(D))

_INFO = plsc.get_sparse_core_info()
_NC = _INFO.num_cores
_NS = _INFO.num_subcores
_NW = _NC * _NS

CHUNK = 128


def _scale_body(t_ref, o_ref):
    o_ref[...] = t_ref[...] * SCALE


def _scaled_table(table):
    rows_per_blk = 5000
    grid = VOCAB // rows_per_blk
    return pl.pallas_call(
        _scale_body,
        out_shape=jax.ShapeDtypeStruct((VOCAB, D), jnp.float32),
        grid=(grid,),
        in_specs=[pl.BlockSpec((rows_per_blk, D), lambda i: (i, 0))],
        out_specs=pl.BlockSpec((rows_per_blk, D), lambda i: (i, 0)),
    )(table)


NBUF = 4
LOOK = 2


@functools.cache
def _gather_kernel(B):
    n_chunks = B // _NW // CHUNK
    per_w = n_chunks * CHUNK
    assert n_chunks % NBUF == 0 and n_chunks > 2 * NBUF
    mesh = plsc.VectorSubcoreMesh(core_axis_name="c", subcore_axis_name="s")

    @functools.partial(
        pl.kernel,
        mesh=mesh,
        out_type=jax.ShapeDtypeStruct((B, D), jnp.float32),
        scratch_types=(
            [pltpu.VMEM((n_chunks, CHUNK), jnp.int32)]
            + [pltpu.VMEM((CHUNK, D), jnp.float32) for _ in range(NBUF)]
            + [pltpu.SemaphoreType.DMA for _ in range(2 * NBUF)]
        ),
    )
    def k(idx_hbm, table_hbm, out_hbm, idx_v, *rest):
        rows = rest[:NBUF]
        gsem = rest[NBUF:2 * NBUF]
        wsem = rest[2 * NBUF:3 * NBUF]
        wid = lax.axis_index("s") * _NC + lax.axis_index("c")
        pltpu.sync_copy(idx_hbm.at[wid], idx_v)
        base = wid * per_w

        def start_gather(c, b):
            pltpu.async_copy(table_hbm.at[idx_v.at[c]], rows[b], gsem[b])

        def wait_gather(c, b):
            pltpu.make_async_copy(table_hbm.at[idx_v.at[c]], rows[b],
                                  gsem[b]).wait()

        def start_write(c, b):
            pltpu.async_copy(rows[b], out_hbm.at[pl.ds(base + c * CHUNK, CHUNK)],
                             wsem[b])

        def wait_write(c, b):
            pltpu.make_async_copy(rows[b],
                                  out_hbm.at[pl.ds(base + c * CHUNK, CHUNK)],
                                  wsem[b]).wait()

        for c in range(LOOK):
            start_gather(c, c)
        for g in range(NBUF):
            wait_gather(g, g)
            start_write(g, g)
            if g + LOOK >= NBUF:
                wait_write(g + LOOK - NBUF, (g + LOOK) % NBUF)
            start_gather(g + LOOK, (g + LOOK) % NBUF)

        def body(outer, carry):
            for b in range(NBUF):
                g = outer * NBUF + b
                wait_gather(g, b)
                start_write(g, b)
                b2 = (b + LOOK) % NBUF
                wait_write(g + LOOK - NBUF, b2)
                start_gather(g + LOOK, b2)
            return carry

        lax.fori_loop(1, n_chunks // NBUF - 1, body, 0)

        for g in range(n_chunks - NBUF, n_chunks):
            b = g % NBUF
            wait_gather(g, b)
            start_write(g, b)
            if g + LOOK < n_chunks:
                b2 = (g + LOOK) % NBUF
                wait_write(g + LOOK - NBUF, b2)
                start_gather(g + LOOK, b2)
        for g in range(n_chunks - NBUF, n_chunks):
            wait_write(g, g % NBUF)

    return k


def kernel(tokens, table):
    b, h = tokens.shape
    B = b * h
    idx = tokens.reshape(_NW, B // _NW // CHUNK, CHUNK)
    scaled = _scaled_table(table)
    out = _gather_kernel(B)(idx, scaled)
    return out.reshape(b, h, D)


# speedup vs baseline: 8.3196x; 1.0030x over previous
import functools
import math

import jax
import jax.numpy as jnp
from jax import lax
from jax.experimental import pallas as pl
from jax.experimental.pallas import tpu as pltpu
from jax.experimental.pallas import tpu_sc as plsc

VOCAB = 100000
D = 128
SCALE = math.sqrt(float(D))

_INFO = plsc.get_sparse_core_info()
_NC = _INFO.num_cores
_NS = _INFO.num_subcores
_NW = _NC * _NS

CHUNK = 128


def _scale_body(t_ref, o_ref):
    o_ref[...] = t_ref[...] * SCALE


def _scaled_table(table):
    rows_per_blk = 5000
    grid = VOCAB // rows_per_blk
    return pl.pallas_call(
        _scale_body,
        out_shape=jax.ShapeDtypeStruct((VOCAB, D), jnp.float32),
        grid=(grid,),
        in_specs=[pl.BlockSpec((rows_per_blk, D), lambda i: (i, 0))],
        out_specs=pl.BlockSpec((rows_per_blk, D), lambda i: (i, 0)),
    )(table)


NBUF = 5
LOOK = 3


@functools.cache
def _gather_kernel(B):
    n_chunks = B // _NW // CHUNK
    per_w = n_chunks * CHUNK
    assert n_chunks % NBUF == 0 and n_chunks > 2 * NBUF
    mesh = plsc.VectorSubcoreMesh(core_axis_name="c", subcore_axis_name="s")

    @functools.partial(
        pl.kernel,
        mesh=mesh,
        out_type=jax.ShapeDtypeStruct((B, D), jnp.float32),
        scratch_types=(
            [pltpu.VMEM((n_chunks, CHUNK), jnp.int32)]
            + [pltpu.VMEM((CHUNK, D), jnp.float32) for _ in range(NBUF)]
            + [pltpu.SemaphoreType.DMA for _ in range(2 * NBUF)]
        ),
    )
    def k(idx_hbm, table_hbm, out_hbm, idx_v, *rest):
        rows = rest[:NBUF]
        gsem = rest[NBUF:2 * NBUF]
        wsem = rest[2 * NBUF:3 * NBUF]
        wid = lax.axis_index("s") * _NC + lax.axis_index("c")
        pltpu.sync_copy(idx_hbm.at[wid], idx_v)
        base = wid * per_w

        def start_gather(c, b):
            pltpu.async_copy(table_hbm.at[idx_v.at[c]], rows[b], gsem[b])

        def wait_gather(c, b):
            pltpu.make_async_copy(table_hbm.at[idx_v.at[c]], rows[b],
                                  gsem[b]).wait()

        def start_write(c, b):
            pltpu.async_copy(rows[b], out_hbm.at[pl.ds(base + c * CHUNK, CHUNK)],
                             wsem[b])

        def wait_write(c, b):
            pltpu.make_async_copy(rows[b],
                                  out_hbm.at[pl.ds(base + c * CHUNK, CHUNK)],
                                  wsem[b]).wait()

        for c in range(LOOK):
            start_gather(c, c)
        for g in range(NBUF):
            wait_gather(g, g)
            start_write(g, g)
            if g + LOOK >= NBUF:
                wait_write(g + LOOK - NBUF, (g + LOOK) % NBUF)
            start_gather(g + LOOK, (g + LOOK) % NBUF)

        def body(outer, carry):
            for b in range(NBUF):
                g = outer * NBUF + b
                wait_gather(g, b)
                start_write(g, b)
                b2 = (b + LOOK) % NBUF
                wait_write(g + LOOK - NBUF, b2)
                start_gather(g + LOOK, b2)
            return carry

        lax.fori_loop(1, n_chunks // NBUF - 1, body, 0)

        for g in range(n_chunks - NBUF, n_chunks):
            b = g % NBUF
            wait_gather(g, b)
            start_write(g, b)
            if g + LOOK < n_chunks:
                b2 = (g + LOOK) % NBUF
                wait_write(g + LOOK - NBUF, b2)
                start_gather(g + LOOK, b2)
        for g in range(n_chunks - NBUF, n_chunks):
            wait_write(g, g % NBUF)

    return k


def kernel(tokens, table):
    b, h = tokens.shape
    B = b * h
    idx = tokens.reshape(_NW, B // _NW // CHUNK, CHUNK)
    scaled = _scaled_table(table)
    out = _gather_kernel(B)(idx, scaled)
    return out.reshape(b, h, D)


# speedup vs baseline: 9.1554x; 1.1005x over previous
import functools
import math

import jax
import jax.numpy as jnp
from jax import lax
from jax.experimental import pallas as pl
from jax.experimental.pallas import tpu as pltpu
from jax.experimental.pallas import tpu_sc as plsc

VOCAB = 100000
D = 128
SCALE = math.sqrt(float(D))

_INFO = plsc.get_sparse_core_info()
_NC = _INFO.num_cores
_NS = _INFO.num_subcores
_NW = _NC * _NS

CHUNK = 128


def _scale_body(t_ref, o_ref):
    o_ref[...] = t_ref[...] * SCALE


def _scaled_table(table):
    rows_per_blk = 5000
    grid = VOCAB // rows_per_blk
    return pl.pallas_call(
        _scale_body,
        out_shape=jax.ShapeDtypeStruct((VOCAB, D), jnp.float32),
        grid=(grid,),
        in_specs=[pl.BlockSpec((rows_per_blk, D), lambda i: (i, 0))],
        out_specs=pl.BlockSpec((rows_per_blk, D), lambda i: (i, 0)),
    )(table)


NBUF = 5
LOOK = 3


@functools.cache
def _gather_kernel(B):
    n_chunks = B // _NW // CHUNK
    per_w = n_chunks * CHUNK
    assert n_chunks % NBUF == 0 and n_chunks > 2 * NBUF
    mesh = plsc.VectorSubcoreMesh(core_axis_name="c", subcore_axis_name="s")

    @functools.partial(
        pl.kernel,
        mesh=mesh,
        out_type=jax.ShapeDtypeStruct((B, D), jnp.float32),
        scratch_types=(
            [pltpu.VMEM((n_chunks, CHUNK), jnp.int32)]
            + [pltpu.VMEM((CHUNK, D), jnp.float32) for _ in range(NBUF)]
            + [pltpu.SemaphoreType.DMA for _ in range(2 * NBUF)]
        ),
    )
    def k(idx_hbm, table_hbm, out_hbm, idx_v, *rest):
        rows = rest[:NBUF]
        gsem = rest[NBUF:2 * NBUF]
        wsem = rest[2 * NBUF:3 * NBUF]
        wid = lax.axis_index("s") * _NC + lax.axis_index("c")
        pltpu.sync_copy(idx_hbm.at[wid], idx_v)
        base = wid * per_w

        def start_gather(c, b):
            pltpu.async_copy(table_hbm.at[idx_v.at[c]], rows[b], gsem[b])

        def wait_gather(c, b):
            pltpu.make_async_copy(table_hbm.at[idx_v.at[c]], rows[b],
                                  gsem[b]).wait()

        def scale_rows(b):
            r = rows[b]
            unroll = 2

            def srow(i, carry):
                for u in range(unroll):
                    for j in range(D // 16):
                        sl = (i * unroll + u, pl.ds(j * 16, 16))
                        r[sl] = r[sl] * SCALE
                return carry

            lax.fori_loop(0, CHUNK // unroll, srow, 0)

        def start_write(c, b):
            pltpu.async_copy(rows[b], out_hbm.at[pl.ds(base + c * CHUNK, CHUNK)],
                             wsem[b])

        def wait_write(c, b):
            pltpu.make_async_copy(rows[b],
                                  out_hbm.at[pl.ds(base + c * CHUNK, CHUNK)],
                                  wsem[b]).wait()

        for c in range(LOOK):
            start_gather(c, c)
        for g in range(NBUF):
            wait_gather(g, g)
            scale_rows(g)
            start_write(g, g)
            if g + LOOK >= NBUF:
                wait_write(g + LOOK - NBUF, (g + LOOK) % NBUF)
            start_gather(g + LOOK, (g + LOOK) % NBUF)

        def body(outer, carry):
            for b in range(NBUF):
                g = outer * NBUF + b
                wait_gather(g, b)
                scale_rows(b)
                start_write(g, b)
                b2 = (b + LOOK) % NBUF
                wait_write(g + LOOK - NBUF, b2)
                start_gather(g + LOOK, b2)
            return carry

        lax.fori_loop(1, n_chunks // NBUF - 1, body, 0)

        for g in range(n_chunks - NBUF, n_chunks):
            b = g % NBUF
            wait_gather(g, b)
            scale_rows(b)
            start_write(g, b)
            if g + LOOK < n_chunks:
                b2 = (g + LOOK) % NBUF
                wait_write(g + LOOK - NBUF, b2)
                start_gather(g + LOOK, b2)
        for g in range(n_chunks - NBUF, n_chunks):
            wait_write(g, g % NBUF)

    return k


def kernel(tokens, table):
    b, h = tokens.shape
    B = b * h
    idx = tokens.reshape(_NW, B // _NW // CHUNK, CHUNK)
    out = _gather_kernel(B)(idx, table)
    return out.reshape(b, h, D)
